# baseline (speedup 1.0000x reference)
import functools

import jax
import jax.numpy as jnp
from jax import lax
from jax.experimental import pallas as pl
from jax.experimental.pallas import tpu as pltpu

_B = 128
_M = 32768
_T = 3
_TOPK = 10
_K = 16
_FLAT = _M * _T
_ROWS = _FLAT // 128
_G = 32
_NG = _ROWS // _G
_NEG = float("-inf")
_BIG = (1 << 30)


def _row_kernel(score_ref, mask_ref, g_ref, actor_ref, pos_ref, head_ref,
                out_ref, s_ref, pos_s, head_s, sem):
    b = pl.program_id(0)

    a = actor_ref[b]
    at = jnp.where((a < 1) | (a > 3), 1, a) - 1

    row_i = lax.broadcasted_iota(jnp.int32, (_ROWS, 128), 0)
    lane_i = lax.broadcasted_iota(jnp.int32, (_ROWS, 128), 1)
    posf = row_i * 128 + lane_i
    gate = (mask_ref[0] != 0) & ((posf % 3) == at)
    s = jnp.where(gate, score_ref[0], _NEG)
    s_ref[...] = s

    P = jnp.max(s.reshape(_NG, _G, 128), axis=1)
    kI = lax.broadcasted_iota(jnp.int32, (_NG, 128), 0)
    lI = lax.broadcasted_iota(jnp.int32, (_NG, 128), 1)
    jG = lax.broadcasted_iota(jnp.int32, (_G, 128), 0)
    lG = lax.broadcasted_iota(jnp.int32, (_G, 128), 1)

    vals = []
    ps = []
    for _ in range(_K):
        v = jnp.max(P)
        pk = jnp.min(jnp.where(P == v, kI * 128 + lI, _BIG))
        k_ = pk // 128
        l_ = pk - k_ * 128
        sub = s_ref[pl.ds(k_ * _G, _G), :]
        j_ = jnp.min(jnp.where((sub == v) & (lG == l_), jG, _BIG))
        p = (k_ * _G + j_) * 128 + l_
        sub = jnp.where((jG == j_) & (lG == l_), _NEG, sub)
        s_ref[pl.ds(k_ * _G, _G), :] = sub
        newmax = jnp.max(jnp.where(lG == l_, sub, _NEG))
        P = jnp.where((kI == k_) & (lI == l_), newmax, P)
        vals.append(v)
        ps.append(p)

    kth = vals[_TOPK - 1]
    m1 = vals[0]

    lane1 = lax.broadcasted_iota(jnp.int32, (1, 128), 1)
    best_z = jnp.float32(_NEG)
    best_m = jnp.int32(0)
    best_v = jnp.float32(0.0)
    sumexp = jnp.float32(0.0)
    for i in range(_K):
        cand = vals[i] >= kth
        finite = vals[i] > _NEG
        sumexp += jnp.where(cand & finite, jnp.exp(vals[i] - m1), 0.0)
        m = ps[i] // 3
        grow = g_ref[0, pl.ds(m // 128, 1), :]
        gv = jnp.max(jnp.where(lane1 == (m % 128), grow, _NEG))
        z = jnp.where(cand, vals[i] + gv, _NEG)
        upd = z > best_z
        best_z = jnp.where(upd, z, best_z)
        best_m = jnp.where(upd, m, best_m)
        best_v = jnp.where(upd, vals[i], best_v)

    logp = best_v - (m1 + jnp.log(sumexp))

    gp = b * _FLAT + best_m * 3
    pbase = jnp.minimum((gp // 128) * 128, _B * _FLAT - 256)
    poff = gp - pbase
    cp = pltpu.make_async_copy(
        pos_ref.at[pl.ds(0, 1), pl.ds(pbase, 256)], pos_s, sem)
    cp.start()
    cp.wait()
    gh = b * _M + best_m
    hbase = (gh // 128) * 128
    hoff = gh - hbase
    ch = pltpu.make_async_copy(
        head_ref.at[pl.ds(0, 1), pl.ds(hbase, 128)], head_s, sem)
    ch.start()
    ch.wait()

    lP = lax.broadcasted_iota(jnp.int32, (1, 256), 1)
    lH = lax.broadcasted_iota(jnp.int32, (1, 128), 1)
    pw = pos_s[...]
    px = jnp.sum(jnp.where(lP == poff, pw, 0.0))
    py = jnp.sum(jnp.where(lP == poff + 1, pw, 0.0))
    pz = jnp.sum(jnp.where(lP == poff + 2, pw, 0.0))
    hh = jnp.sum(jnp.where(lH == hoff, head_s[...], 0.0))

    lane8 = lax.broadcasted_iota(jnp.int32, (8, 128), 1)
    vec = jnp.where(
        lane8 == 0, px,
        jnp.where(lane8 == 1, py,
                  jnp.where(lane8 == 2, pz,
                            jnp.where(lane8 == 3, hh,
                                      jnp.where(lane8 == 4, logp, 0.0)))))
    out_ref[0] = vec.astype(jnp.float32)


@jax.jit
def kernel(fake_map_feat_score, map_valid_mask, map_position, map_heading,
           actor_type):
    score_r = fake_map_feat_score.reshape(_B, _ROWS, 128)
    mask_r = map_valid_mask.reshape(_B, _ROWS, 128).astype(jnp.int8)
    u = jax.random.uniform(jax.random.key(42), (_B, _M),
                           minval=1e-9, maxval=1.0)
    gumbel = (-jnp.log(-jnp.log(u))).reshape(_B, _M // 128, 128)
    actor = actor_type.astype(jnp.int32)

    out = pl.pallas_call(
        _row_kernel,
        grid=(_B,),
        in_specs=[
            pl.BlockSpec((1, _ROWS, 128), lambda b: (b, 0, 0)),
            pl.BlockSpec((1, _ROWS, 128), lambda b: (b, 0, 0)),
            pl.BlockSpec((1, _M // 128, 128), lambda b: (b, 0, 0)),
            pl.BlockSpec(memory_space=pltpu.SMEM),
            pl.BlockSpec(memory_space=pl.ANY),
            pl.BlockSpec(memory_space=pl.ANY),
        ],
        out_specs=pl.BlockSpec((1, 8, 128), lambda b: (b, 0, 0)),
        out_shape=jax.ShapeDtypeStruct((_B, 8, 128), jnp.float32),
        scratch_shapes=[
            pltpu.VMEM((_ROWS, 128), jnp.float32),
            pltpu.VMEM((1, 256), jnp.float32),
            pltpu.VMEM((1, 128), jnp.float32),
            pltpu.SemaphoreType.DMA,
        ],
    )(score_r, mask_r, gumbel, actor, map_position.reshape(1, _B * _FLAT),
      map_heading.reshape(1, _B * _M))
    return out[:, 0, :5]
